# baseline (speedup 1.0000x reference)
import functools

import jax
import jax.numpy as jnp
from jax import lax
from jax.experimental import pallas as pl
from jax.experimental.pallas import tpu as pltpu
from jax.experimental.pallas import tpu_sc as plsc

CODE_DIM = 64
DIM = 256
IDX_CHUNK = 64
NBUF = 7


@functools.lru_cache(maxsize=None)
def _make_project(vocab: int):
    blk = 4096

    def mm(cbt_ref, w_ref, b_ref, out_ref):
        out_ref[...] = (
            lax.dot_general(
                cbt_ref[...],
                w_ref[...],
                (((0,), (0,)), ((), ())),
                preferred_element_type=jnp.float32,
            )
            + b_ref[...]
        )

    call = pl.pallas_call(
        mm,
        grid=(vocab // blk,),
        in_specs=[
            pl.BlockSpec((CODE_DIM, blk), lambda i: (0, i)),
            pl.BlockSpec((CODE_DIM, DIM), lambda i: (0, 0)),
            pl.BlockSpec((1, DIM), lambda i: (0, 0)),
        ],
        out_specs=pl.BlockSpec((blk, DIM), lambda i: (i, 0)),
        out_shape=jax.ShapeDtypeStruct((vocab, DIM), jnp.float32),
    )

    def run(codebook, W, bias):
        return call(codebook.T, W, bias.reshape(1, DIM))

    return run


@functools.lru_cache(maxsize=None)
def _make_gather(bsz: int, tlen: int):
    ntok = bsz * tlen
    info = plsc.get_sparse_core_info()
    nc = info.num_cores
    nw = nc * info.num_subcores
    per_w = ntok // nw
    nchunk = per_w // IDX_CHUNK
    wper_row = tlen // per_w
    mesh = plsc.VectorSubcoreMesh(core_axis_name="c", subcore_axis_name="s")

    @functools.partial(
        pl.kernel,
        mesh=mesh,
        out_type=jax.ShapeDtypeStruct((ntok, DIM), jnp.float32),
        scratch_types=[
            pltpu.VMEM((per_w,), jnp.int32),
            [pltpu.VMEM((IDX_CHUNK, DIM), jnp.float32)] * NBUF,
            pltpu.SemaphoreType.DMA,
            [pltpu.SemaphoreType.DMA] * NBUF,
        ],
    )
    def gather_kernel(idx_hbm, proj_hbm, out_hbm, idx_v, bufs, gsem, ssems):
        wid = lax.axis_index("s") * nc + lax.axis_index("c")
        base = wid * per_w
        row = wid // wper_row
        col = (wid % wper_row) * per_w
        pltpu.sync_copy(idx_hbm.at[row, pl.ds(col, per_w)], idx_v)

        def gather(c):
            return pltpu.async_copy(
                proj_hbm.at[idx_v.at[pl.ds(c * IDX_CHUNK, IDX_CHUNK)]],
                bufs[c % NBUF],
                gsem,
            )

        def scatter(c):
            return pltpu.async_copy(
                bufs[c % NBUF],
                out_hbm.at[pl.ds(base + c * IDX_CHUNK, IDX_CHUNK)],
                ssems[c % NBUF],
            )

        depth = min(NBUF - 1, nchunk)
        gpend = [gather(c) for c in range(depth)]
        spend = [None] * NBUF
        for j in range(nchunk):
            gpend[j % depth].wait()
            spend[j % NBUF] = scatter(j)
            nxt = j + depth
            if nxt < nchunk:
                if spend[nxt % NBUF] is not None:
                    spend[nxt % NBUF].wait()
                gpend[nxt % depth] = gather(nxt)
        for p in spend:
            if p is not None:
                p.wait()

    return gather_kernel


def kernel(indices, codebook, W, b):
    bsz, tlen = indices.shape
    vocab = codebook.shape[0]
    proj = _make_project(vocab)(codebook, W, b)
    out = _make_gather(bsz, tlen)(indices.astype(jnp.int32), proj)
    return out.reshape(bsz, tlen, DIM)


# speedup vs baseline: 1.0199x; 1.0199x over previous
import functools

import jax
import jax.numpy as jnp
from jax import lax
from jax.experimental import pallas as pl
from jax.experimental.pallas import tpu as pltpu
from jax.experimental.pallas import tpu_sc as plsc

CODE_DIM = 64
DIM = 256
IDX_CHUNK = 64
NBUF = 7


@functools.lru_cache(maxsize=None)
def _make_project(vocab: int):
    blk = 4096

    def mm(cbt_ref, w_ref, b_ref, out_ref):
        out_ref[...] = (
            lax.dot_general(
                cbt_ref[...],
                w_ref[...],
                (((0,), (0,)), ((), ())),
                preferred_element_type=jnp.float32,
            )
            + b_ref[...]
        )

    call = pl.pallas_call(
        mm,
        grid=(vocab // blk,),
        in_specs=[
            pl.BlockSpec((CODE_DIM, blk), lambda i: (0, i)),
            pl.BlockSpec((CODE_DIM, DIM), lambda i: (0, 0)),
            pl.BlockSpec((1, DIM), lambda i: (0, 0)),
        ],
        out_specs=pl.BlockSpec((blk, DIM), lambda i: (i, 0)),
        out_shape=jax.ShapeDtypeStruct((vocab, DIM), jnp.float32),
    )

    def run(codebook, W, bias):
        return call(codebook.T, W, bias.reshape(1, DIM))

    return run


@functools.lru_cache(maxsize=None)
def _make_gather(bsz: int, tlen: int):
    ntok = bsz * tlen
    info = plsc.get_sparse_core_info()
    nc = info.num_cores
    nw = nc * info.num_subcores
    per_w = ntok // nw
    nchunk = per_w // IDX_CHUNK
    wper_row = tlen // per_w
    mesh = plsc.VectorSubcoreMesh(core_axis_name="c", subcore_axis_name="s")

    @functools.partial(
        pl.kernel,
        mesh=mesh,
        out_type=jax.ShapeDtypeStruct((ntok, DIM), jnp.float32),
        scratch_types=[
            pltpu.VMEM((per_w,), jnp.int32),
            [pltpu.VMEM((IDX_CHUNK, DIM), jnp.float32)] * NBUF,
            [pltpu.SemaphoreType.DMA] * NBUF,
            [pltpu.SemaphoreType.DMA] * NBUF,
        ],
    )
    def gather_kernel(idx_hbm, proj_hbm, out_hbm, idx_v, bufs, gsems, ssems):
        wid = lax.axis_index("s") * nc + lax.axis_index("c")
        base = wid * per_w
        row = wid // wper_row
        col = (wid % wper_row) * per_w
        pltpu.sync_copy(idx_hbm.at[row, pl.ds(col, per_w)], idx_v)

        def gather(c):
            return pltpu.async_copy(
                proj_hbm.at[idx_v.at[pl.ds(c * IDX_CHUNK, IDX_CHUNK)]],
                bufs[c % NBUF],
                gsems[c % NBUF],
            )

        def scatter(c):
            return pltpu.async_copy(
                bufs[c % NBUF],
                out_hbm.at[pl.ds(base + c * IDX_CHUNK, IDX_CHUNK)],
                ssems[c % NBUF],
            )

        depth = min(NBUF - 1, nchunk)
        gpend = [gather(c) for c in range(depth)]
        spend = [None] * NBUF
        for j in range(nchunk):
            gpend[j % depth].wait()
            spend[j % NBUF] = scatter(j)
            nxt = j + depth
            if nxt < nchunk:
                if spend[nxt % NBUF] is not None:
                    spend[nxt % NBUF].wait()
                gpend[nxt % depth] = gather(nxt)
        for p in spend:
            if p is not None:
                p.wait()

    return gather_kernel


def kernel(indices, codebook, W, b):
    bsz, tlen = indices.shape
    vocab = codebook.shape[0]
    proj = _make_project(vocab)(codebook, W, b)
    out = _make_gather(bsz, tlen)(indices.astype(jnp.int32), proj)
    return out.reshape(bsz, tlen, DIM)


# speedup vs baseline: 1.0256x; 1.0056x over previous
import functools

import jax
import jax.numpy as jnp
from jax import lax
from jax.experimental import pallas as pl
from jax.experimental.pallas import tpu as pltpu
from jax.experimental.pallas import tpu_sc as plsc

CODE_DIM = 64
DIM = 256
IDX_CHUNK = 32
NBUF = 14


@functools.lru_cache(maxsize=None)
def _make_project(vocab: int):
    blk = 4096

    def mm(cbt_ref, w_ref, b_ref, out_ref):
        out_ref[...] = (
            lax.dot_general(
                cbt_ref[...],
                w_ref[...],
                (((0,), (0,)), ((), ())),
                preferred_element_type=jnp.float32,
            )
            + b_ref[...]
        )

    call = pl.pallas_call(
        mm,
        grid=(vocab // blk,),
        in_specs=[
            pl.BlockSpec((CODE_DIM, blk), lambda i: (0, i)),
            pl.BlockSpec((CODE_DIM, DIM), lambda i: (0, 0)),
            pl.BlockSpec((1, DIM), lambda i: (0, 0)),
        ],
        out_specs=pl.BlockSpec((blk, DIM), lambda i: (i, 0)),
        out_shape=jax.ShapeDtypeStruct((vocab, DIM), jnp.float32),
    )

    def run(codebook, W, bias):
        return call(codebook.T, W, bias.reshape(1, DIM))

    return run


@functools.lru_cache(maxsize=None)
def _make_gather(bsz: int, tlen: int):
    ntok = bsz * tlen
    info = plsc.get_sparse_core_info()
    nc = info.num_cores
    nw = nc * info.num_subcores
    per_w = ntok // nw
    nchunk = per_w // IDX_CHUNK
    wper_row = tlen // per_w
    mesh = plsc.VectorSubcoreMesh(core_axis_name="c", subcore_axis_name="s")

    @functools.partial(
        pl.kernel,
        mesh=mesh,
        out_type=jax.ShapeDtypeStruct((ntok, DIM), jnp.float32),
        scratch_types=[
            pltpu.VMEM((per_w,), jnp.int32),
            [pltpu.VMEM((IDX_CHUNK, DIM), jnp.float32)] * NBUF,
            [pltpu.SemaphoreType.DMA] * NBUF,
            [pltpu.SemaphoreType.DMA] * NBUF,
        ],
    )
    def gather_kernel(idx_hbm, proj_hbm, out_hbm, idx_v, bufs, gsems, ssems):
        wid = lax.axis_index("s") * nc + lax.axis_index("c")
        base = wid * per_w
        row = wid // wper_row
        col = (wid % wper_row) * per_w
        pltpu.sync_copy(idx_hbm.at[row, pl.ds(col, per_w)], idx_v)

        def gather(c):
            return pltpu.async_copy(
                proj_hbm.at[idx_v.at[pl.ds(c * IDX_CHUNK, IDX_CHUNK)]],
                bufs[c % NBUF],
                gsems[c % NBUF],
            )

        def scatter(c):
            return pltpu.async_copy(
                bufs[c % NBUF],
                out_hbm.at[pl.ds(base + c * IDX_CHUNK, IDX_CHUNK)],
                ssems[c % NBUF],
            )

        depth = min(NBUF - 1, nchunk)
        gpend = [gather(c) for c in range(depth)]
        spend = [None] * NBUF
        for j in range(nchunk):
            gpend[j % depth].wait()
            spend[j % NBUF] = scatter(j)
            nxt = j + depth
            if nxt < nchunk:
                if spend[nxt % NBUF] is not None:
                    spend[nxt % NBUF].wait()
                gpend[nxt % depth] = gather(nxt)
        for p in spend:
            if p is not None:
                p.wait()

    return gather_kernel


def kernel(indices, codebook, W, b):
    bsz, tlen = indices.shape
    vocab = codebook.shape[0]
    proj = _make_project(vocab)(codebook, W, b)
    out = _make_gather(bsz, tlen)(indices.astype(jnp.int32), proj)
    return out.reshape(bsz, tlen, DIM)
